# Initial kernel scaffold; baseline (speedup 1.0000x reference)
import jax
import jax.numpy as jnp
from jax.experimental import pallas as pl


def kernel(x, pos, spectral_filter, bn_gamma, bn_beta, W1, b1, W2, b2, Wf, bf):
    raise NotImplementedError("write your pallas kernel here")


# speedup vs baseline: 9.3909x; 9.3909x over previous
import functools

import jax
import jax.numpy as jnp
from jax import lax
from jax.experimental import pallas as pl
from jax.experimental.pallas import tpu as pltpu
from jax.experimental.pallas import tpu_sc as plsc

B, C, N, K, RED = 2, 128, 4096, 16, 4
BN_PTS = B * N
HI = jax.lax.Precision.HIGHEST

BLKQ = 256


def _topk_body(posq_ref, posa_ref, idx_ref):
    b = pl.program_id(0)
    posq = posq_ref[0]
    posa = posa_ref[0]
    posq_r = posq.astype(jnp.bfloat16).astype(jnp.float32)
    posa_r = posa.astype(jnp.bfloat16).astype(jnp.float32)
    inner = lax.dot_general(posq_r, posa_r, (((0,), (0,)), ((), ())),
                            precision=HI, preferred_element_type=jnp.float32)
    sqq = jnp.sum(posq * posq, axis=0)[:, None]
    sqa = jnp.sum(posa * posa, axis=0)[None, :]
    d = sqq + sqa - 2.0 * inner
    col = lax.broadcasted_iota(jnp.int32, (BLKQ, N), 1)
    lane = lax.broadcasted_iota(jnp.int32, (BLKQ, K), 1)
    idxbuf = jnp.zeros((BLKQ, K), jnp.int32)
    for t in range(K):
        m = jnp.min(d, axis=1, keepdims=True)
        am = jnp.min(jnp.where(d == m, col, jnp.int32(N)), axis=1,
                     keepdims=True)
        flat = am + (b * K + t) * N
        idxbuf = jnp.where(lane == t, flat, idxbuf)
        d = jnp.where(col == am, jnp.float32(jnp.inf), d)
    idx_ref[0] = idxbuf


def _topk(pos):
    return pl.pallas_call(
        _topk_body,
        grid=(B, N // BLKQ),
        in_specs=[
            pl.BlockSpec((1, 3, BLKQ), lambda b, q: (b, 0, q)),
            pl.BlockSpec((1, 3, N), lambda b, q: (b, 0, 0)),
        ],
        out_specs=pl.BlockSpec((1, BLKQ, K), lambda b, q: (b, q, 0)),
        out_shape=jax.ShapeDtypeStruct((B, N, K), jnp.int32),
    )(pos, pos)


BLKZ = 2048


def _ztab_body(x_ref, f_ref, z_ref):
    xb = x_ref[0]
    fk = f_ref[0]
    z_ref[...] = lax.dot_general(xb, fk, (((0,), (0,)), ((), ())),
                                 precision=HI,
                                 preferred_element_type=jnp.float32)


def _ztab(x, sf):
    nz = N // BLKZ
    return pl.pallas_call(
        _ztab_body,
        grid=(B, K, nz),
        in_specs=[
            pl.BlockSpec((1, C, BLKZ), lambda b, k, z: (b, 0, z)),
            pl.BlockSpec((1, C, C), lambda b, k, z: (k, 0, 0)),
        ],
        out_specs=pl.BlockSpec((BLKZ, C), lambda b, k, z: ((b * K + k) * nz + z, 0)),
        out_shape=jax.ShapeDtypeStruct((B * K * N, C), jnp.float32),
    )(x, sf)


NW = 32
PTS_PER_W = BN_PTS // NW
PTS_PER_CHUNK = 8
CHUNKS = PTS_PER_W // PTS_PER_CHUNK


def _sc_gather_body(idx_hbm, z_hbm, out_hbm, idx_v, rows_v, acc_v, sem):
    wid = lax.axis_index("s") * 2 + lax.axis_index("c")
    base_pt = wid * PTS_PER_W

    def chunk(j, carry):
        pb = base_pt + j * PTS_PER_CHUNK
        pltpu.sync_copy(idx_hbm.at[pl.ds(pb * K, PTS_PER_CHUNK * K)], idx_v)
        pltpu.async_copy(z_hbm.at[idx_v], rows_v, sem).wait()
        for p in range(PTS_PER_CHUNK):
            for l in range(8):
                acc = rows_v[p * K, pl.ds(l * 16, 16)]
                for r in range(1, K):
                    acc = acc + rows_v[p * K + r, pl.ds(l * 16, 16)]
                acc_v[p, pl.ds(l * 16, 16)] = acc
        pltpu.sync_copy(acc_v, out_hbm.at[pl.ds(pb, PTS_PER_CHUNK)])
        return carry

    lax.fori_loop(0, CHUNKS, chunk, 0)


@functools.cache
def _sc_gather_fn():
    return functools.partial(
        pl.kernel,
        out_type=jax.ShapeDtypeStruct((BN_PTS, C), jnp.float32),
        mesh=plsc.VectorSubcoreMesh(core_axis_name="c", subcore_axis_name="s"),
        scratch_types=[
            pltpu.VMEM((PTS_PER_CHUNK * K,), jnp.int32),
            pltpu.VMEM((PTS_PER_CHUNK * K, C), jnp.float32),
            pltpu.VMEM((PTS_PER_CHUNK, C), jnp.float32),
            pltpu.SemaphoreType.DMA,
        ],
    )(_sc_gather_body)


def _sc_gather(idx_flat, z):
    return _sc_gather_fn()(idx_flat, z)


BLK4 = 1024


def _stats_body(x_ref, sf_ref, acc_ref, st_ref):
    first = (pl.program_id(0) == 0) & (pl.program_id(1) == 0)
    xb = x_ref[0]
    fsum = jnp.sum(sf_ref[...], axis=0)
    center = lax.dot_general(xb, fsum, (((0,), (0,)), ((), ())),
                             precision=HI, preferred_element_type=jnp.float32)
    spec = acc_ref[...] - center
    s1 = jnp.sum(spec, axis=0, keepdims=True)
    s2 = jnp.sum(spec * spec, axis=0, keepdims=True)
    row = lax.broadcasted_iota(jnp.int32, (8, C), 0)
    contrib = jnp.where(row == 0, jnp.broadcast_to(s1, (8, C)),
                        jnp.where(row == 1, jnp.broadcast_to(s2, (8, C)),
                                  jnp.float32(0.0)))

    @pl.when(first)
    def _():
        st_ref[...] = contrib

    @pl.when(jnp.logical_not(first))
    def _():
        st_ref[...] = st_ref[...] + contrib


def _stats(x, sf, acc):
    nb = N // BLK4
    return pl.pallas_call(
        _stats_body,
        grid=(B, nb),
        in_specs=[
            pl.BlockSpec((1, C, BLK4), lambda b, n: (b, 0, n)),
            pl.BlockSpec((K, C, C), lambda b, n: (0, 0, 0)),
            pl.BlockSpec((BLK4, C), lambda b, n: (b * nb + n, 0)),
        ],
        out_specs=pl.BlockSpec((8, C), lambda b, n: (0, 0)),
        out_shape=jax.ShapeDtypeStruct((8, C), jnp.float32),
    )(x, sf, acc)


BLK5 = 1024


def _final_body(x_ref, sf_ref, acc_ref, st_ref, g_ref, be_ref, w1_ref,
                b1_ref, w2_ref, b2_ref, wf_ref, bf_ref, out_ref):
    xb = x_ref[0]
    fsum = jnp.sum(sf_ref[...], axis=0)
    center = lax.dot_general(xb, fsum, (((0,), (0,)), ((), ())),
                             precision=HI, preferred_element_type=jnp.float32)
    spec = acc_ref[...] - center
    inv = jnp.float32(1.0 / BN_PTS)
    mean = st_ref[0:1, :] * inv
    var = st_ref[1:2, :] * inv - mean * mean
    rstd = lax.rsqrt(var + 1e-5)
    xn = (spec - mean) * (rstd * g_ref[0:1, :]) + be_ref[0:1, :]
    a1 = lax.dot_general(xn, w1_ref[...], (((1,), (1,)), ((), ())),
                         precision=HI, preferred_element_type=jnp.float32)
    a1 = jnp.maximum(a1 + b1_ref[0:1, :], 0.0)
    a2 = lax.dot_general(a1, w2_ref[...], (((1,), (1,)), ((), ())),
                         precision=HI, preferred_element_type=jnp.float32)
    attn = jax.nn.sigmoid(a2 + b2_ref[0:1, :])
    xs = xn * attn
    wfa = wf_ref[:, :C]
    wfb = wf_ref[:, C:]
    outa = lax.dot_general(xb, wfa, (((0,), (1,)), ((), ())),
                           precision=HI, preferred_element_type=jnp.float32)
    outb = lax.dot_general(xs, wfb, (((1,), (1,)), ((), ())),
                           precision=HI, preferred_element_type=jnp.float32)
    out_ref[...] = outa + outb + bf_ref[0:1, :]


def _final(x, sf, acc, st, gamma, beta, w1, b1, w2, b2, wf, bf):
    nb = N // BLK5
    return pl.pallas_call(
        _final_body,
        grid=(B, nb),
        in_specs=[
            pl.BlockSpec((1, C, BLK5), lambda b, n: (b, 0, n)),
            pl.BlockSpec((K, C, C), lambda b, n: (0, 0, 0)),
            pl.BlockSpec((BLK5, C), lambda b, n: (b * nb + n, 0)),
            pl.BlockSpec((8, C), lambda b, n: (0, 0)),
            pl.BlockSpec((8, C), lambda b, n: (0, 0)),
            pl.BlockSpec((8, C), lambda b, n: (0, 0)),
            pl.BlockSpec((C // RED, C), lambda b, n: (0, 0)),
            pl.BlockSpec((8, C // RED), lambda b, n: (0, 0)),
            pl.BlockSpec((C, C // RED), lambda b, n: (0, 0)),
            pl.BlockSpec((8, C), lambda b, n: (0, 0)),
            pl.BlockSpec((C, 2 * C), lambda b, n: (0, 0)),
            pl.BlockSpec((8, C), lambda b, n: (0, 0)),
        ],
        out_specs=pl.BlockSpec((BLK5, C), lambda b, n: (b * nb + n, 0)),
        out_shape=jax.ShapeDtypeStruct((BN_PTS, C), jnp.float32),
    )(x, sf, acc, st, gamma, beta, w1, b1, w2, b2, wf, bf)


def kernel(x, pos, spectral_filter, bn_gamma, bn_beta, W1, b1, W2, b2, Wf, bf):
    idx = _topk(pos)
    z = _ztab(x, spectral_filter)
    idx_flat = idx.reshape(B * N * K)
    acc = _sc_gather(idx_flat, z)
    st = _stats(x, spectral_filter, acc)
    g8 = jnp.tile(bn_gamma[None, :], (8, 1))
    be8 = jnp.tile(bn_beta[None, :], (8, 1))
    b18 = jnp.tile(b1[None, :], (8, 1))
    b28 = jnp.tile(b2[None, :], (8, 1))
    bf8 = jnp.tile(bf[None, :], (8, 1))
    outt = _final(x, spectral_filter, acc, st, g8, be8, W1, b18, W2, b28,
                  Wf, bf8)
    return outt.reshape(B, N, C).transpose(0, 2, 1)


# speedup vs baseline: 9.7455x; 1.0378x over previous
import functools

import jax
import jax.numpy as jnp
from jax import lax
from jax.experimental import pallas as pl
from jax.experimental.pallas import tpu as pltpu
from jax.experimental.pallas import tpu_sc as plsc

B, C, N, K, RED = 2, 128, 4096, 16, 4
BN_PTS = B * N
HI = jax.lax.Precision.HIGHEST

BLKQ = 256


def _topk_body(posq_ref, posa_ref, idx_ref):
    b = pl.program_id(0)
    posq = posq_ref[0]
    posa = posa_ref[0]
    posq_r = posq.astype(jnp.bfloat16).astype(jnp.float32)
    posa_r = posa.astype(jnp.bfloat16).astype(jnp.float32)
    inner = lax.dot_general(posq_r, posa_r, (((0,), (0,)), ((), ())),
                            precision=HI, preferred_element_type=jnp.float32)
    sqq = jnp.sum(posq * posq, axis=0)[:, None]
    sqa = jnp.sum(posa * posa, axis=0)[None, :]
    d = sqq + sqa - 2.0 * inner
    col = lax.broadcasted_iota(jnp.int32, (BLKQ, N), 1)
    lane = lax.broadcasted_iota(jnp.int32, (BLKQ, K), 1)
    idxbuf = jnp.zeros((BLKQ, K), jnp.int32)
    for t in range(K):
        m = jnp.min(d, axis=1, keepdims=True)
        am = jnp.min(jnp.where(d == m, col, jnp.int32(N)), axis=1,
                     keepdims=True)
        flat = am + (b * K + t) * N
        idxbuf = jnp.where(lane == t, flat, idxbuf)
        d = jnp.where(col == am, jnp.float32(jnp.inf), d)
    idx_ref[0] = idxbuf


def _topk(pos):
    return pl.pallas_call(
        _topk_body,
        grid=(B, N // BLKQ),
        in_specs=[
            pl.BlockSpec((1, 3, BLKQ), lambda b, q: (b, 0, q)),
            pl.BlockSpec((1, 3, N), lambda b, q: (b, 0, 0)),
        ],
        out_specs=pl.BlockSpec((1, BLKQ, K), lambda b, q: (b, q, 0)),
        out_shape=jax.ShapeDtypeStruct((B, N, K), jnp.int32),
    )(pos, pos)


BLKZ = 1024
HI3 = jax.lax.Precision.HIGH


def _ztab_body(x_ref, f_ref, z_ref):
    xb = x_ref[0]
    for k in range(K):
        z_ref[k] = lax.dot_general(xb, f_ref[k], (((0,), (0,)), ((), ())),
                                   precision=HI,
                                   preferred_element_type=jnp.float32)


def _ztab(x, sf):
    out = pl.pallas_call(
        _ztab_body,
        grid=(B, N // BLKZ),
        in_specs=[
            pl.BlockSpec((1, C, BLKZ), lambda b, z: (b, 0, z)),
            pl.BlockSpec((K, C, C), lambda b, z: (0, 0, 0)),
        ],
        out_specs=pl.BlockSpec((K, BLKZ, C), lambda b, z: (b, z, 0)),
        out_shape=jax.ShapeDtypeStruct((B * K, N, C), jnp.float32),
    )(x, sf)
    return out.reshape(B * K * N, C)


NW = 32
PTS_PER_W = BN_PTS // NW
PTS_PER_CHUNK = 8
CHUNKS = PTS_PER_W // PTS_PER_CHUNK


def _sc_gather_body(idx_hbm, z_hbm, out_hbm, idx_v, rows_v, acc_v, sem):
    wid = lax.axis_index("s") * 2 + lax.axis_index("c")
    base_pt = wid * PTS_PER_W

    def chunk(j, carry):
        pb = base_pt + j * PTS_PER_CHUNK
        pltpu.sync_copy(idx_hbm.at[pl.ds(pb * K, PTS_PER_CHUNK * K)], idx_v)
        pltpu.async_copy(z_hbm.at[idx_v], rows_v, sem).wait()
        for p in range(PTS_PER_CHUNK):
            for l in range(8):
                acc = rows_v[p * K, pl.ds(l * 16, 16)]
                for r in range(1, K):
                    acc = acc + rows_v[p * K + r, pl.ds(l * 16, 16)]
                acc_v[p, pl.ds(l * 16, 16)] = acc
        pltpu.sync_copy(acc_v, out_hbm.at[pl.ds(pb, PTS_PER_CHUNK)])
        return carry

    lax.fori_loop(0, CHUNKS, chunk, 0)


@functools.cache
def _sc_gather_fn():
    return functools.partial(
        pl.kernel,
        out_type=jax.ShapeDtypeStruct((BN_PTS, C), jnp.float32),
        mesh=plsc.VectorSubcoreMesh(core_axis_name="c", subcore_axis_name="s"),
        scratch_types=[
            pltpu.VMEM((PTS_PER_CHUNK * K,), jnp.int32),
            pltpu.VMEM((PTS_PER_CHUNK * K, C), jnp.float32),
            pltpu.VMEM((PTS_PER_CHUNK, C), jnp.float32),
            pltpu.SemaphoreType.DMA,
        ],
    )(_sc_gather_body)


def _sc_gather(idx_flat, z):
    return _sc_gather_fn()(idx_flat, z)


BLK4 = 1024


def _stats_body(x_ref, sf_ref, acc_ref, st_ref):
    first = (pl.program_id(0) == 0) & (pl.program_id(1) == 0)
    xb = x_ref[0]
    fsum = jnp.sum(sf_ref[...], axis=0)
    center = lax.dot_general(xb, fsum, (((0,), (0,)), ((), ())),
                             precision=HI, preferred_element_type=jnp.float32)
    spec = acc_ref[...] - center
    s1 = jnp.sum(spec, axis=0, keepdims=True)
    s2 = jnp.sum(spec * spec, axis=0, keepdims=True)
    row = lax.broadcasted_iota(jnp.int32, (8, C), 0)
    contrib = jnp.where(row == 0, jnp.broadcast_to(s1, (8, C)),
                        jnp.where(row == 1, jnp.broadcast_to(s2, (8, C)),
                                  jnp.float32(0.0)))

    @pl.when(first)
    def _():
        st_ref[...] = contrib

    @pl.when(jnp.logical_not(first))
    def _():
        st_ref[...] = st_ref[...] + contrib


def _stats(x, sf, acc):
    nb = N // BLK4
    return pl.pallas_call(
        _stats_body,
        grid=(B, nb),
        in_specs=[
            pl.BlockSpec((1, C, BLK4), lambda b, n: (b, 0, n)),
            pl.BlockSpec((K, C, C), lambda b, n: (0, 0, 0)),
            pl.BlockSpec((BLK4, C), lambda b, n: (b * nb + n, 0)),
        ],
        out_specs=pl.BlockSpec((8, C), lambda b, n: (0, 0)),
        out_shape=jax.ShapeDtypeStruct((8, C), jnp.float32),
    )(x, sf, acc)


BLK5 = 1024


def _final_body(x_ref, sf_ref, acc_ref, st_ref, g_ref, be_ref, w1_ref,
                b1_ref, w2_ref, b2_ref, wf_ref, bf_ref, out_ref):
    xb = x_ref[0]
    fsum = jnp.sum(sf_ref[...], axis=0)
    center = lax.dot_general(xb, fsum, (((0,), (0,)), ((), ())),
                             precision=HI, preferred_element_type=jnp.float32)
    spec = acc_ref[...] - center
    inv = jnp.float32(1.0 / BN_PTS)
    mean = st_ref[0:1, :] * inv
    var = st_ref[1:2, :] * inv - mean * mean
    rstd = lax.rsqrt(var + 1e-5)
    xn = (spec - mean) * (rstd * g_ref[0:1, :]) + be_ref[0:1, :]
    a1 = lax.dot_general(xn, w1_ref[...], (((1,), (1,)), ((), ())),
                         precision=HI, preferred_element_type=jnp.float32)
    a1 = jnp.maximum(a1 + b1_ref[0:1, :], 0.0)
    a2 = lax.dot_general(a1, w2_ref[...], (((1,), (1,)), ((), ())),
                         precision=HI, preferred_element_type=jnp.float32)
    attn = jax.nn.sigmoid(a2 + b2_ref[0:1, :])
    xs = xn * attn
    wfa = wf_ref[:, :C]
    wfb = wf_ref[:, C:]
    outa = lax.dot_general(xb, wfa, (((0,), (1,)), ((), ())),
                           precision=HI, preferred_element_type=jnp.float32)
    outb = lax.dot_general(xs, wfb, (((1,), (1,)), ((), ())),
                           precision=HI, preferred_element_type=jnp.float32)
    out_ref[...] = outa + outb + bf_ref[0:1, :]


def _final(x, sf, acc, st, gamma, beta, w1, b1, w2, b2, wf, bf):
    nb = N // BLK5
    return pl.pallas_call(
        _final_body,
        grid=(B, nb),
        in_specs=[
            pl.BlockSpec((1, C, BLK5), lambda b, n: (b, 0, n)),
            pl.BlockSpec((K, C, C), lambda b, n: (0, 0, 0)),
            pl.BlockSpec((BLK5, C), lambda b, n: (b * nb + n, 0)),
            pl.BlockSpec((8, C), lambda b, n: (0, 0)),
            pl.BlockSpec((8, C), lambda b, n: (0, 0)),
            pl.BlockSpec((8, C), lambda b, n: (0, 0)),
            pl.BlockSpec((C // RED, C), lambda b, n: (0, 0)),
            pl.BlockSpec((8, C // RED), lambda b, n: (0, 0)),
            pl.BlockSpec((C, C // RED), lambda b, n: (0, 0)),
            pl.BlockSpec((8, C), lambda b, n: (0, 0)),
            pl.BlockSpec((C, 2 * C), lambda b, n: (0, 0)),
            pl.BlockSpec((8, C), lambda b, n: (0, 0)),
        ],
        out_specs=pl.BlockSpec((BLK5, C), lambda b, n: (b * nb + n, 0)),
        out_shape=jax.ShapeDtypeStruct((BN_PTS, C), jnp.float32),
    )(x, sf, acc, st, gamma, beta, w1, b1, w2, b2, wf, bf)


def kernel(x, pos, spectral_filter, bn_gamma, bn_beta, W1, b1, W2, b2, Wf, bf):
    idx = _topk(pos)
    z = _ztab(x, spectral_filter)
    idx_flat = idx.reshape(B * N * K)
    acc = _sc_gather(idx_flat, z)
    st = _stats(x, spectral_filter, acc)
    g8 = jnp.tile(bn_gamma[None, :], (8, 1))
    be8 = jnp.tile(bn_beta[None, :], (8, 1))
    b18 = jnp.tile(b1[None, :], (8, 1))
    b28 = jnp.tile(b2[None, :], (8, 1))
    bf8 = jnp.tile(bf[None, :], (8, 1))
    outt = _final(x, spectral_filter, acc, st, g8, be8, W1, b18, W2, b28,
                  Wf, bf8)
    return outt.reshape(B, N, C).transpose(0, 2, 1)


# speedup vs baseline: 10.5855x; 1.0862x over previous
import functools

import jax
import jax.numpy as jnp
from jax import lax
from jax.experimental import pallas as pl
from jax.experimental.pallas import tpu as pltpu
from jax.experimental.pallas import tpu_sc as plsc

B, C, N, K, RED = 2, 128, 4096, 16, 4
BN_PTS = B * N
HI = jax.lax.Precision.HIGHEST

BLKQ = 256


def _topk_body(posq_ref, posa_ref, idx_ref):
    b = pl.program_id(0)
    posq = posq_ref[0]
    posa = posa_ref[0]
    posq_r = posq.astype(jnp.bfloat16).astype(jnp.float32)
    posa_r = posa.astype(jnp.bfloat16).astype(jnp.float32)
    inner = lax.dot_general(posq_r, posa_r, (((0,), (0,)), ((), ())),
                            precision=HI, preferred_element_type=jnp.float32)
    sqq = jnp.sum(posq * posq, axis=0)[:, None]
    sqa = jnp.sum(posa * posa, axis=0)[None, :]
    d = sqq + sqa - 2.0 * inner
    col = lax.broadcasted_iota(jnp.int32, (BLKQ, N), 1)
    lane = lax.broadcasted_iota(jnp.int32, (BLKQ, K), 1)
    idxbuf = jnp.zeros((BLKQ, K), jnp.int32)
    for t in range(K):
        m = jnp.min(d, axis=1, keepdims=True)
        mask = d == m
        am = jnp.min(jnp.where(mask, col, jnp.int32(N)), axis=1,
                     keepdims=True)
        flat = am + (b * K + t) * N
        idxbuf = jnp.where(lane == t, flat, idxbuf)
        d = jnp.where(mask, jnp.float32(jnp.inf), d)
    idx_ref[0] = idxbuf


def _topk(pos):
    return pl.pallas_call(
        _topk_body,
        grid=(B, N // BLKQ),
        in_specs=[
            pl.BlockSpec((1, 3, BLKQ), lambda b, q: (b, 0, q)),
            pl.BlockSpec((1, 3, N), lambda b, q: (b, 0, 0)),
        ],
        out_specs=pl.BlockSpec((1, BLKQ, K), lambda b, q: (b, q, 0)),
        out_shape=jax.ShapeDtypeStruct((B, N, K), jnp.int32),
    )(pos, pos)


BLKZ = 1024
HI3 = jax.lax.Precision.HIGH


def _ztab_body(x_ref, f_ref, z_ref):
    xb = x_ref[0]
    for k in range(K):
        z_ref[k] = lax.dot_general(xb, f_ref[k], (((0,), (0,)), ((), ())),
                                   precision=HI,
                                   preferred_element_type=jnp.float32)


def _ztab(x, sf):
    out = pl.pallas_call(
        _ztab_body,
        grid=(B, N // BLKZ),
        in_specs=[
            pl.BlockSpec((1, C, BLKZ), lambda b, z: (b, 0, z)),
            pl.BlockSpec((K, C, C), lambda b, z: (0, 0, 0)),
        ],
        out_specs=pl.BlockSpec((K, BLKZ, C), lambda b, z: (b, z, 0)),
        out_shape=jax.ShapeDtypeStruct((B * K, N, C), jnp.float32),
    )(x, sf)
    return out.reshape(B * K * N, C)


NW = 32
PTS_PER_W = BN_PTS // NW
PTS_PER_CHUNK = 8
CHUNKS = PTS_PER_W // PTS_PER_CHUNK


def _sc_gather_body(idx_hbm, z_hbm, out_hbm, idx_v, rows_v, acc_v, sem):
    wid = lax.axis_index("s") * 2 + lax.axis_index("c")
    base_pt = wid * PTS_PER_W

    def chunk(j, carry):
        pb = base_pt + j * PTS_PER_CHUNK
        pltpu.sync_copy(idx_hbm.at[pl.ds(pb * K, PTS_PER_CHUNK * K)], idx_v)
        pltpu.async_copy(z_hbm.at[idx_v], rows_v, sem).wait()
        for p in range(PTS_PER_CHUNK):
            for l in range(8):
                acc = rows_v[p * K, pl.ds(l * 16, 16)]
                for r in range(1, K):
                    acc = acc + rows_v[p * K + r, pl.ds(l * 16, 16)]
                acc_v[p, pl.ds(l * 16, 16)] = acc
        pltpu.sync_copy(acc_v, out_hbm.at[pl.ds(pb, PTS_PER_CHUNK)])
        return carry

    lax.fori_loop(0, CHUNKS, chunk, 0)


@functools.cache
def _sc_gather_fn():
    return functools.partial(
        pl.kernel,
        out_type=jax.ShapeDtypeStruct((BN_PTS, C), jnp.float32),
        mesh=plsc.VectorSubcoreMesh(core_axis_name="c", subcore_axis_name="s"),
        scratch_types=[
            pltpu.VMEM((PTS_PER_CHUNK * K,), jnp.int32),
            pltpu.VMEM((PTS_PER_CHUNK * K, C), jnp.float32),
            pltpu.VMEM((PTS_PER_CHUNK, C), jnp.float32),
            pltpu.SemaphoreType.DMA,
        ],
    )(_sc_gather_body)


def _sc_gather(idx_flat, z):
    return _sc_gather_fn()(idx_flat, z)


BLK4 = 1024


def _stats_body(x_ref, sf_ref, acc_ref, st_ref):
    first = (pl.program_id(0) == 0) & (pl.program_id(1) == 0)
    xb = x_ref[0]
    fsum = jnp.sum(sf_ref[...], axis=0)
    center = lax.dot_general(xb, fsum, (((0,), (0,)), ((), ())),
                             precision=HI, preferred_element_type=jnp.float32)
    spec = acc_ref[...] - center
    s1 = jnp.sum(spec, axis=0, keepdims=True)
    s2 = jnp.sum(spec * spec, axis=0, keepdims=True)
    row = lax.broadcasted_iota(jnp.int32, (8, C), 0)
    contrib = jnp.where(row == 0, jnp.broadcast_to(s1, (8, C)),
                        jnp.where(row == 1, jnp.broadcast_to(s2, (8, C)),
                                  jnp.float32(0.0)))

    @pl.when(first)
    def _():
        st_ref[...] = contrib

    @pl.when(jnp.logical_not(first))
    def _():
        st_ref[...] = st_ref[...] + contrib


def _stats(x, sf, acc):
    nb = N // BLK4
    return pl.pallas_call(
        _stats_body,
        grid=(B, nb),
        in_specs=[
            pl.BlockSpec((1, C, BLK4), lambda b, n: (b, 0, n)),
            pl.BlockSpec((K, C, C), lambda b, n: (0, 0, 0)),
            pl.BlockSpec((BLK4, C), lambda b, n: (b * nb + n, 0)),
        ],
        out_specs=pl.BlockSpec((8, C), lambda b, n: (0, 0)),
        out_shape=jax.ShapeDtypeStruct((8, C), jnp.float32),
    )(x, sf, acc)


BLK5 = 1024


def _final_body(x_ref, sf_ref, acc_ref, st_ref, g_ref, be_ref, w1_ref,
                b1_ref, w2_ref, b2_ref, wf_ref, bf_ref, out_ref):
    xb = x_ref[0]
    fsum = jnp.sum(sf_ref[...], axis=0)
    center = lax.dot_general(xb, fsum, (((0,), (0,)), ((), ())),
                             precision=HI, preferred_element_type=jnp.float32)
    spec = acc_ref[...] - center
    inv = jnp.float32(1.0 / BN_PTS)
    mean = st_ref[0:1, :] * inv
    var = st_ref[1:2, :] * inv - mean * mean
    rstd = lax.rsqrt(var + 1e-5)
    xn = (spec - mean) * (rstd * g_ref[0:1, :]) + be_ref[0:1, :]
    a1 = lax.dot_general(xn, w1_ref[...], (((1,), (1,)), ((), ())),
                         precision=HI, preferred_element_type=jnp.float32)
    a1 = jnp.maximum(a1 + b1_ref[0:1, :], 0.0)
    a2 = lax.dot_general(a1, w2_ref[...], (((1,), (1,)), ((), ())),
                         precision=HI, preferred_element_type=jnp.float32)
    attn = jax.nn.sigmoid(a2 + b2_ref[0:1, :])
    xs = xn * attn
    wfa = wf_ref[:, :C]
    wfb = wf_ref[:, C:]
    outa = lax.dot_general(xb, wfa, (((0,), (1,)), ((), ())),
                           precision=HI, preferred_element_type=jnp.float32)
    outb = lax.dot_general(xs, wfb, (((1,), (1,)), ((), ())),
                           precision=HI, preferred_element_type=jnp.float32)
    out_ref[...] = outa + outb + bf_ref[0:1, :]


def _final(x, sf, acc, st, gamma, beta, w1, b1, w2, b2, wf, bf):
    nb = N // BLK5
    return pl.pallas_call(
        _final_body,
        grid=(B, nb),
        in_specs=[
            pl.BlockSpec((1, C, BLK5), lambda b, n: (b, 0, n)),
            pl.BlockSpec((K, C, C), lambda b, n: (0, 0, 0)),
            pl.BlockSpec((BLK5, C), lambda b, n: (b * nb + n, 0)),
            pl.BlockSpec((8, C), lambda b, n: (0, 0)),
            pl.BlockSpec((8, C), lambda b, n: (0, 0)),
            pl.BlockSpec((8, C), lambda b, n: (0, 0)),
            pl.BlockSpec((C // RED, C), lambda b, n: (0, 0)),
            pl.BlockSpec((8, C // RED), lambda b, n: (0, 0)),
            pl.BlockSpec((C, C // RED), lambda b, n: (0, 0)),
            pl.BlockSpec((8, C), lambda b, n: (0, 0)),
            pl.BlockSpec((C, 2 * C), lambda b, n: (0, 0)),
            pl.BlockSpec((8, C), lambda b, n: (0, 0)),
        ],
        out_specs=pl.BlockSpec((BLK5, C), lambda b, n: (b * nb + n, 0)),
        out_shape=jax.ShapeDtypeStruct((BN_PTS, C), jnp.float32),
    )(x, sf, acc, st, gamma, beta, w1, b1, w2, b2, wf, bf)


def kernel(x, pos, spectral_filter, bn_gamma, bn_beta, W1, b1, W2, b2, Wf, bf):
    idx = _topk(pos)
    z = _ztab(x, spectral_filter)
    idx_flat = idx.reshape(B * N * K)
    acc = _sc_gather(idx_flat, z)
    st = _stats(x, spectral_filter, acc)
    g8 = jnp.tile(bn_gamma[None, :], (8, 1))
    be8 = jnp.tile(bn_beta[None, :], (8, 1))
    b18 = jnp.tile(b1[None, :], (8, 1))
    b28 = jnp.tile(b2[None, :], (8, 1))
    bf8 = jnp.tile(bf[None, :], (8, 1))
    outt = _final(x, spectral_filter, acc, st, g8, be8, W1, b18, W2, b28,
                  Wf, bf8)
    return outt.reshape(B, N, C).transpose(0, 2, 1)
